# Initial kernel scaffold; baseline (speedup 1.0000x reference)
import jax
import jax.numpy as jnp
from jax.experimental import pallas as pl


def kernel(indices, table):
    raise NotImplementedError("write your pallas kernel here")


# speedup vs baseline: 1.6758x; 1.6758x over previous
import functools

import jax
import jax.numpy as jnp
from jax import lax
from jax.experimental import pallas as pl
from jax.experimental.pallas import tpu as pltpu
from jax.experimental.pallas import tpu_sc as plsc

_L = 16
_NW = 32
_CH = 128


def kernel(indices, table):
    B, S = indices.shape
    V, D = table.shape
    N = B * S
    assert N % (_NW * _CH) == 0 and D % _L == 0
    n_ch = N // (_NW * _CH)
    idx2d = indices.reshape(N // _CH, _CH)

    mesh = plsc.VectorSubcoreMesh(core_axis_name="c", subcore_axis_name="s")

    @functools.partial(
        pl.kernel,
        mesh=mesh,
        out_type=jax.ShapeDtypeStruct((N, D), jnp.float32),
        compiler_params=pltpu.CompilerParams(use_tc_tiling_on_sc=False),
        scratch_types=[
            pltpu.VMEM((n_ch, _CH), jnp.int32),
            pltpu.VMEM((_CH, D), jnp.float32),
            pltpu.VMEM((_L,), jnp.int32),
            pltpu.SemaphoreType.DMA,
        ],
    )
    def _emb(idx_hbm, table_hbm, out_hbm, idx_v, rows_v, cnt_v, sem):
        wid = lax.axis_index("s") * 2 + lax.axis_index("c")
        pltpu.sync_copy(idx_hbm.at[pl.ds(wid * n_ch, n_ch)], idx_v)

        def chunk(j, carry):
            pltpu.async_copy(table_hbm.at[idx_v.at[j]], rows_v, sem).wait()

            def cnt_group(i, accv):
                iv = idx_v[j, pl.ds(i * _L, _L)]
                return accv + (1 - jnp.minimum(iv, 1))

            accv = lax.fori_loop(
                0, _CH // _L, cnt_group, jnp.zeros((_L,), jnp.int32)
            )
            nzero = accv[0]
            for _k in range(1, _L):
                nzero = nzero + accv[_k]

            @pl.when(nzero > 0)
            def _fix():
                zeros = jnp.zeros((_L,), jnp.float32)

                def fix_group(i, carry2):
                    iv = idx_v[j, pl.ds(i * _L, _L)]
                    for l in range(_L):
                        val = iv[l]

                        @pl.when(val == 0)
                        def _zrow(l=l):
                            row = i * _L + l
                            for c in range(D // _L):
                                rows_v[row, pl.ds(c * _L, _L)] = zeros

                    return carry2

                lax.fori_loop(0, _CH // _L, fix_group, 0)

            pltpu.sync_copy(
                rows_v, out_hbm.at[pl.ds((wid * n_ch + j) * _CH, _CH)]
            )
            return carry

        lax.fori_loop(0, n_ch, chunk, 0)

    out = _emb(idx2d, table)
    return out.reshape(B, S, D)


# speedup vs baseline: 1.8760x; 1.1195x over previous
import functools

import jax
import jax.numpy as jnp
from jax import lax
from jax.experimental import pallas as pl
from jax.experimental.pallas import tpu as pltpu
from jax.experimental.pallas import tpu_sc as plsc

_L = 16
_NW = 32
_CH = 128
_R = 4


def kernel(indices, table):
    B, S = indices.shape
    V, D = table.shape
    N = B * S
    assert N % (_NW * _CH) == 0 and D % _L == 0
    n_ch = N // (_NW * _CH)
    assert n_ch % _R == 0
    idx2d = indices.reshape(N // _CH, _CH)

    mesh = plsc.VectorSubcoreMesh(core_axis_name="c", subcore_axis_name="s")

    @functools.partial(
        pl.kernel,
        mesh=mesh,
        out_type=jax.ShapeDtypeStruct((N, D), jnp.float32),
        compiler_params=pltpu.CompilerParams(use_tc_tiling_on_sc=False),
        scratch_types=[
            pltpu.VMEM((n_ch, _CH), jnp.int32),
            pltpu.VMEM((_R, _CH, D), jnp.float32),
        ]
        + [pltpu.SemaphoreType.DMA] * _R,
    )
    def _emb(idx_hbm, table_hbm, out_hbm, idx_v, rows_v, *sems):
        wid = lax.axis_index("s") * 2 + lax.axis_index("c")
        pltpu.sync_copy(idx_hbm.at[pl.ds(wid * n_ch, n_ch)], idx_v)

        def start_gather(j, b):
            pltpu.async_copy(table_hbm.at[idx_v.at[j]], rows_v.at[b], sems[b])

        def wait_gather(j, b):
            pltpu.make_async_copy(
                table_hbm.at[idx_v.at[j]], rows_v.at[b], sems[b]
            ).wait()

        def process(j, b):
            def cnt_group(i, accv):
                iv = idx_v[j, pl.ds(i * _L, _L)]
                return accv + (1 - jnp.minimum(iv, 1))

            accv = lax.fori_loop(
                0, _CH // _L, cnt_group, jnp.zeros((_L,), jnp.int32)
            )
            nzero = accv[0]
            for _k in range(1, _L):
                nzero = nzero + accv[_k]

            @pl.when(nzero > 0)
            def _fix():
                zeros = jnp.zeros((_L,), jnp.float32)

                def fix_group(i, carry2):
                    iv = idx_v[j, pl.ds(i * _L, _L)]
                    for l in range(_L):
                        val = iv[l]

                        @pl.when(val == 0)
                        def _zrow(l=l):
                            row = i * _L + l
                            for c in range(D // _L):
                                rows_v[b, row, pl.ds(c * _L, _L)] = zeros

                    return carry2

                lax.fori_loop(0, _CH // _L, fix_group, 0)

            pltpu.sync_copy(
                rows_v.at[b], out_hbm.at[pl.ds((wid * n_ch + j) * _CH, _CH)]
            )

        for b in range(_R):
            start_gather(b, b)

        n_steps = n_ch // _R

        def step_body(step, carry):
            for b in range(_R):
                j = step * _R + b
                wait_gather(j, b)
                process(j, b)

                @pl.when(step < n_steps - 1)
                def _next(j=j, b=b):
                    start_gather(j + _R, b)

            return carry

        lax.fori_loop(0, n_steps, step_body, 0)

    out = _emb(idx2d, table)
    return out.reshape(B, S, D)
